# baseline (speedup 1.0000x reference)
import jax
import jax.numpy as jnp
from jax.experimental import pallas as pl
from jax.experimental.pallas import tpu as pltpu


def _prep_body(Wvis_ref, Wself_ref, Wnbr_ref, bvis_ref, bconv_ref,
               AB_ref, bias_ref, emb, hid):
    Wsv = Wself_ref[emb:, :]
    Wnv = Wnbr_ref[emb:, :]
    Wvis = Wvis_ref[...]
    AB_ref[:, :hid] = jnp.dot(Wvis, Wsv, preferred_element_type=jnp.float32)
    AB_ref[:, hid:] = jnp.dot(Wvis, Wnv, preferred_element_type=jnp.float32)
    bias_ref[...] = (jnp.dot(bvis_ref[...], Wsv + Wnv,
                             preferred_element_type=jnp.float32)
                     + bconv_ref[...])


def _main_body(v1_ref, v2_ref, c1_ref, c2_ref, tbl_ref, WW_ref, AB_ref,
               bias_ref, W1_ref, b1_ref, w2_ref, b2_ref, out_ref,
               ncat_pad, hid):
    r = v1_ref.shape[0]
    ids1 = c1_ref[0]
    ids2 = c2_ref[0]
    iota = jax.lax.broadcasted_iota(jnp.int16, (r, ncat_pad), 1)
    one = jnp.bfloat16(1.0)
    zero = jnp.bfloat16(0.0)
    oh1 = jnp.where(ids1 == iota, one, zero)
    oh2 = jnp.where(ids2 == iota, one, zero)
    tbl = tbl_ref[...]
    e1 = jnp.dot(oh1, tbl, preferred_element_type=jnp.float32)
    e2 = jnp.dot(oh2, tbl, preferred_element_type=jnp.float32)
    WW = WW_ref[...]
    Q1 = jnp.dot(e1, WW, preferred_element_type=jnp.float32)
    Q2 = jnp.dot(e2, WW, preferred_element_type=jnp.float32)
    AB = AB_ref[...]
    P1 = jnp.dot(v1_ref[...], AB, preferred_element_type=jnp.float32)
    P2 = jnp.dot(v2_ref[...], AB, preferred_element_type=jnp.float32)
    b = bias_ref[...]
    u = Q1[:, :hid] + Q2[:, hid:] + P1[:, :hid] + P2[:, hid:] + b
    w = Q2[:, :hid] + Q1[:, hid:] + P2[:, :hid] + P1[:, hid:] + b
    pooled = (jnp.maximum(u, 0.0) + jnp.maximum(w, 0.0)) * 0.5
    hid_act = jnp.maximum(
        jnp.dot(pooled, W1_ref[...], preferred_element_type=jnp.float32)
        + b1_ref[...], 0.0)
    score = (jnp.dot(hid_act, w2_ref[...], preferred_element_type=jnp.float32)
             + b2_ref[...])
    out_ref[0] = score


def kernel(cat_id1, visual1, cat_id2, visual2, cat_table, W_vis, b_vis,
           W_self, W_nbr, b_conv, W1, b1, w2, b2):
    n, vis = visual1.shape
    ncat, emb = cat_table.shape
    hid = W1.shape[0]
    ncat_pad = ((ncat + 127) // 128) * 128
    R = 1000
    nb = n // R

    c1 = cat_id1.astype(jnp.int16).reshape(nb, R, 1)
    c2 = cat_id2.astype(jnp.int16).reshape(nb, R, 1)
    tbl = jnp.pad(cat_table, ((0, ncat_pad - ncat), (0, 0))).astype(jnp.bfloat16)
    WW = jnp.concatenate([W_self[:emb], W_nbr[:emb]], axis=1)

    AB, bias = pl.pallas_call(
        lambda *refs: _prep_body(*refs, emb=emb, hid=hid),
        out_shape=[
            jax.ShapeDtypeStruct((vis, 2 * hid), jnp.float32),
            jax.ShapeDtypeStruct((1, hid), jnp.float32),
        ],
    )(W_vis, W_self, W_nbr, b_vis.reshape(1, emb), b_conv.reshape(1, hid))

    grid = (nb,)
    out = pl.pallas_call(
        lambda *refs: _main_body(*refs, ncat_pad=ncat_pad, hid=hid),
        grid=grid,
        in_specs=[
            pl.BlockSpec((R, vis), lambda i: (i, 0)),
            pl.BlockSpec((R, vis), lambda i: (i, 0)),
            pl.BlockSpec((1, R, 1), lambda i: (i, 0, 0)),
            pl.BlockSpec((1, R, 1), lambda i: (i, 0, 0)),
            pl.BlockSpec((ncat_pad, emb), lambda i: (0, 0)),
            pl.BlockSpec((emb, 2 * hid), lambda i: (0, 0)),
            pl.BlockSpec((vis, 2 * hid), lambda i: (0, 0)),
            pl.BlockSpec((1, hid), lambda i: (0, 0)),
            pl.BlockSpec((hid, hid), lambda i: (0, 0)),
            pl.BlockSpec((1, hid), lambda i: (0, 0)),
            pl.BlockSpec((hid, 1), lambda i: (0, 0)),
            pl.BlockSpec((1, 1), lambda i: (0, 0)),
        ],
        out_specs=pl.BlockSpec((1, R, 1), lambda i: (i, 0, 0)),
        out_shape=jax.ShapeDtypeStruct((nb, R, 1), jnp.float32),
    )(visual1, visual2, c1, c2, tbl, WW, AB, bias, W1,
      b1.reshape(1, hid), w2.reshape(hid, 1), b2.reshape(1, 1))

    return out.reshape(n)


# speedup vs baseline: 1.9783x; 1.9783x over previous
import jax
import jax.numpy as jnp
from jax.experimental import pallas as pl
from jax.experimental.pallas import tpu as pltpu


def _prep_body(Wvis_ref, Wself_ref, Wnbr_ref, bvis_ref, bconv_ref,
               AB_ref, bias_ref, emb, hid):
    Wsv = Wself_ref[emb:, :]
    Wnv = Wnbr_ref[emb:, :]
    Wvis = Wvis_ref[...]
    AB_ref[:, :hid] = jnp.dot(Wvis, Wsv, preferred_element_type=jnp.float32)
    AB_ref[:, hid:] = jnp.dot(Wvis, Wnv, preferred_element_type=jnp.float32)
    bias_ref[...] = (jnp.dot(bvis_ref[...], Wsv + Wnv,
                             preferred_element_type=jnp.float32)
                     + bconv_ref[...])


def _main_body(v1_ref, v2_ref, c1_ref, c2_ref, tblT_ref, WW_ref, AB_ref,
               bias_ref, W1_ref, b1_ref, w2_ref, b2_ref, out_ref,
               ncat_pad, hid):
    r = v1_ref.shape[0]
    ids1 = c1_ref[0]
    ids2 = c2_ref[0]
    iota = jax.lax.broadcasted_iota(jnp.int32, (ncat_pad, r), 0)
    oh1T = (ids1 == iota).astype(jnp.bfloat16)
    oh2T = (ids2 == iota).astype(jnp.bfloat16)
    tblT = tblT_ref[...]
    e1 = jnp.dot(tblT, oh1T, preferred_element_type=jnp.float32).T
    e2 = jnp.dot(tblT, oh2T, preferred_element_type=jnp.float32).T
    WW = WW_ref[...]
    Q1 = jnp.dot(e1, WW, preferred_element_type=jnp.float32)
    Q2 = jnp.dot(e2, WW, preferred_element_type=jnp.float32)
    AB = AB_ref[...]
    P1 = jnp.dot(v1_ref[...], AB, preferred_element_type=jnp.float32)
    P2 = jnp.dot(v2_ref[...], AB, preferred_element_type=jnp.float32)
    b = bias_ref[...]
    u = Q1[:, :hid] + Q2[:, hid:] + P1[:, :hid] + P2[:, hid:] + b
    w = Q2[:, :hid] + Q1[:, hid:] + P2[:, :hid] + P1[:, hid:] + b
    pooled = (jnp.maximum(u, 0.0) + jnp.maximum(w, 0.0)) * 0.5
    hid_act = jnp.maximum(
        jnp.dot(pooled, W1_ref[...], preferred_element_type=jnp.float32)
        + b1_ref[...], 0.0)
    score = (jnp.dot(hid_act, w2_ref[...], preferred_element_type=jnp.float32)
             + b2_ref[...])
    out_ref[0] = score.T


def kernel(cat_id1, visual1, cat_id2, visual2, cat_table, W_vis, b_vis,
           W_self, W_nbr, b_conv, W1, b1, w2, b2):
    n, vis = visual1.shape
    ncat, emb = cat_table.shape
    hid = W1.shape[0]
    ncat_pad = ((ncat + 127) // 128) * 128
    R = 1000
    nb = n // R

    c1 = cat_id1.astype(jnp.int32).reshape(nb, 1, R)
    c2 = cat_id2.astype(jnp.int32).reshape(nb, 1, R)
    tblT = jnp.pad(cat_table, ((0, ncat_pad - ncat), (0, 0))).astype(jnp.bfloat16).T
    WW = jnp.concatenate([W_self[:emb], W_nbr[:emb]], axis=1)

    AB, bias = pl.pallas_call(
        lambda *refs: _prep_body(*refs, emb=emb, hid=hid),
        out_shape=[
            jax.ShapeDtypeStruct((vis, 2 * hid), jnp.float32),
            jax.ShapeDtypeStruct((1, hid), jnp.float32),
        ],
    )(W_vis, W_self, W_nbr, b_vis.reshape(1, emb), b_conv.reshape(1, hid))

    grid = (nb,)
    out = pl.pallas_call(
        lambda *refs: _main_body(*refs, ncat_pad=ncat_pad, hid=hid),
        grid=grid,
        in_specs=[
            pl.BlockSpec((R, vis), lambda i: (i, 0)),
            pl.BlockSpec((R, vis), lambda i: (i, 0)),
            pl.BlockSpec((1, 1, R), lambda i: (i, 0, 0)),
            pl.BlockSpec((1, 1, R), lambda i: (i, 0, 0)),
            pl.BlockSpec((emb, ncat_pad), lambda i: (0, 0)),
            pl.BlockSpec((emb, 2 * hid), lambda i: (0, 0)),
            pl.BlockSpec((vis, 2 * hid), lambda i: (0, 0)),
            pl.BlockSpec((1, hid), lambda i: (0, 0)),
            pl.BlockSpec((hid, hid), lambda i: (0, 0)),
            pl.BlockSpec((1, hid), lambda i: (0, 0)),
            pl.BlockSpec((hid, 1), lambda i: (0, 0)),
            pl.BlockSpec((1, 1), lambda i: (0, 0)),
        ],
        out_specs=pl.BlockSpec((1, 1, R), lambda i: (i, 0, 0)),
        out_shape=jax.ShapeDtypeStruct((nb, 1, R), jnp.float32),
    )(visual1, visual2, c1, c2, tblT, WW, AB, bias, W1,
      b1.reshape(1, hid), w2.reshape(hid, 1), b2.reshape(1, 1))

    return out.reshape(n)
